# baseline (speedup 1.0000x reference)
import functools

import jax
import jax.numpy as jnp
from jax import lax
from jax.experimental import pallas as pl
from jax.experimental.pallas import tpu as pltpu
from jax.experimental.pallas import tpu_sc as plsc

N_NODES = 10000
N_EDGES = 320000
D_IN = 128
D_HID = 128
D_OUT = 40

NC = 2
NS = 16
NW = NC * NS
LANES = 16

NPAD = 10112
CHUNK = 32
EPT = 10240
E_PAD = NW * EPT
EPT_RAW = N_EDGES // NW
ROWS_PT = NPAD // NS

_mesh = plsc.VectorSubcoreMesh(
    core_axis_name="c", subcore_axis_name="s", num_cores=NC, num_subcores=NS
)


@functools.partial(
    pl.kernel,
    out_type=jax.ShapeDtypeStruct((NW, NPAD), jnp.float32),
    mesh=_mesh,
    scratch_types=[
        pltpu.VMEM((EPT_RAW,), jnp.int32),
        pltpu.VMEM((NPAD,), jnp.float32),
    ],
    compiler_params=pltpu.CompilerParams(needs_layout_passes=False),
)
def _sc_degree(dst_hbm, out_hbm, idx_v, deg_v):
    wid = lax.axis_index("s") * NC + lax.axis_index("c")
    pltpu.sync_copy(dst_hbm.at[pl.ds(wid * EPT_RAW, EPT_RAW)], idx_v)
    zero16 = jnp.zeros((LANES,), jnp.float32)

    def zbody(i, carry):
        deg_v[pl.ds(i * LANES, LANES)] = zero16
        return carry

    lax.fori_loop(0, NPAD // LANES, zbody, 0, unroll=8)
    one16 = jnp.ones((LANES,), jnp.float32)

    def body(i, carry):
        idx = idx_v[pl.ds(i * LANES, LANES)]
        plsc.addupdate_scatter(deg_v, [idx], one16)
        return carry

    lax.fori_loop(0, EPT_RAW // LANES, body, 0, unroll=5)
    pltpu.sync_copy(deg_v, out_hbm.at[wid])


NBUF = 4
CPT = EPT // CHUNK


@functools.partial(
    pl.kernel,
    out_type=jax.ShapeDtypeStruct((NC, NPAD, D_HID), jnp.float32),
    mesh=_mesh,
    scratch_types=[
        pltpu.VMEM((EPT,), jnp.int32),
        pltpu.VMEM((CPT, CHUNK), jnp.int32),
        pltpu.VMEM((NBUF, CHUNK, D_HID), jnp.float32),
        pltpu.VMEM_SHARED((NPAD, D_HID), jnp.float32),
    ]
    + [pltpu.SemaphoreType.DMA] * NBUF,
    compiler_params=pltpu.CompilerParams(use_tc_tiling_on_sc=False),
)
def _sc_scatter(src_hbm, dst_hbm, hn_hbm, z_hbm, out_hbm,
                srcv, dstv, rows, agg, *sems):
    cid = lax.axis_index("c")
    sid = lax.axis_index("s")
    wid = sid * NC + cid
    pltpu.sync_copy(src_hbm.at[pl.ds(wid * EPT, EPT)], srcv)
    pltpu.sync_copy(dst_hbm.at[pl.ds(wid * CPT, CPT)], dstv)

    def sidx(j):
        return srcv.at[pl.ds(j * CHUNK, CHUNK)]
    stripe = pl.ds(sid * ROWS_PT, ROWS_PT)

    @pl.when(cid == 0)
    def _():
        pltpu.sync_copy(hn_hbm.at[stripe], agg.at[stripe])

    @pl.when(cid == 1)
    def _():
        pltpu.sync_copy(z_hbm, agg.at[stripe])

    for b in range(NBUF - 1):
        pltpu.async_copy(hn_hbm.at[sidx(b)], rows.at[b], sems[b])
    plsc.subcore_barrier()

    def group(g, carry):
        j0 = g * NBUF
        for b in range(NBUF):
            j = j0 + b
            jn = j + NBUF - 1
            bn = (b - 1) % NBUF

            @pl.when(jn < CPT)
            def _():
                pltpu.async_copy(hn_hbm.at[sidx(jn)], rows.at[bn], sems[bn])

            pltpu.make_async_copy(hn_hbm.at[sidx(0)], rows.at[b], sems[b]).wait()
            pltpu.sync_copy(rows.at[b], agg.at[dstv.at[j]], add=True)
        return carry

    lax.fori_loop(0, CPT // NBUF, group, 0)
    plsc.subcore_barrier()
    pltpu.sync_copy(agg.at[pl.ds(sid * ROWS_PT, ROWS_PT)],
                    out_hbm.at[cid, pl.ds(sid * ROWS_PT, ROWS_PT)])


def _tc_hn_body(degp_ref, x_ref, w_ref, hn_ref, dis_ref):
    ones = jnp.ones((NW, 1), jnp.float32)
    deg_col = lax.dot_general(degp_ref[...], ones, (((0,), (0,)), ((), ())),
                              preferred_element_type=jnp.float32) + 1.0
    dis = lax.rsqrt(deg_col)
    h = jnp.dot(x_ref[...], w_ref[...], preferred_element_type=jnp.float32)
    hn_ref[:N_NODES] = h * dis[:N_NODES]
    hn_ref[N_NODES:] = jnp.zeros((NPAD - N_NODES, D_HID), jnp.float32)
    dis_ref[...] = dis


def _tc_final_body(p_ref, dis_ref, w2_ref, b1_ref, b2_ref, o_ref):
    s = p_ref[0, :N_NODES] + p_ref[1, :N_NODES]
    a = s * dis_ref[:N_NODES] + b1_ref[...][None, :]
    r = jnp.maximum(a, 0.0)
    o = jnp.dot(r, w2_ref[...], preferred_element_type=jnp.float32)
    o = o + b2_ref[...][None, :]
    m = jnp.max(o, axis=-1, keepdims=True)
    ex = jnp.exp(o - m)
    lse = jnp.log(jnp.sum(ex, axis=-1, keepdims=True)) + m
    o_ref[...] = o - lse


def kernel(x, edge_index, batch, W1, b1, W2, b2):
    del batch
    src = edge_index[0].astype(jnp.int32)
    dst = edge_index[1].astype(jnp.int32)
    n_pad_e = E_PAD - N_EDGES
    pad_idx = N_NODES + (jnp.arange(n_pad_e, dtype=jnp.int32) % (NPAD - N_NODES))
    src_p2 = jnp.concatenate([src, pad_idx])
    dst_p2 = jnp.concatenate([dst, pad_idx]).reshape(NW * (EPT // CHUNK), CHUNK)

    degp = _sc_degree(dst)

    hn, dis_col = pl.pallas_call(
        _tc_hn_body,
        out_shape=(jax.ShapeDtypeStruct((NPAD, D_HID), jnp.float32),
                   jax.ShapeDtypeStruct((NPAD, 1), jnp.float32)),
    )(degp, x, W1)

    zeros = jnp.zeros((ROWS_PT, D_HID), jnp.float32)
    p = _sc_scatter(src_p2, dst_p2, hn, zeros)

    out = pl.pallas_call(
        _tc_final_body,
        out_shape=jax.ShapeDtypeStruct((N_NODES, D_OUT), jnp.float32),
    )(p, dis_col, W2, b1, b2)
    return out


# speedup vs baseline: 1.0076x; 1.0076x over previous
import functools

import jax
import jax.numpy as jnp
from jax import lax
from jax.experimental import pallas as pl
from jax.experimental.pallas import tpu as pltpu
from jax.experimental.pallas import tpu_sc as plsc

N_NODES = 10000
N_EDGES = 320000
D_IN = 128
D_HID = 128
D_OUT = 40

NC = 2
NS = 16
NW = NC * NS
LANES = 16

NPAD = 10112
CHUNK = 32
EPT = 10240
E_PAD = NW * EPT
EPT_RAW = N_EDGES // NW
ROWS_PT = NPAD // NS

_mesh = plsc.VectorSubcoreMesh(
    core_axis_name="c", subcore_axis_name="s", num_cores=NC, num_subcores=NS
)


@functools.partial(
    pl.kernel,
    out_type=jax.ShapeDtypeStruct((NW, NPAD), jnp.float32),
    mesh=_mesh,
    scratch_types=[
        pltpu.VMEM((EPT_RAW,), jnp.int32),
        pltpu.VMEM((NPAD,), jnp.float32),
    ],
    compiler_params=pltpu.CompilerParams(needs_layout_passes=False),
)
def _sc_degree(dst_hbm, out_hbm, idx_v, deg_v):
    wid = lax.axis_index("s") * NC + lax.axis_index("c")
    pltpu.sync_copy(dst_hbm.at[pl.ds(wid * EPT_RAW, EPT_RAW)], idx_v)
    zero16 = jnp.zeros((LANES,), jnp.float32)

    def zbody(i, carry):
        deg_v[pl.ds(i * LANES, LANES)] = zero16
        return carry

    lax.fori_loop(0, NPAD // LANES, zbody, 0, unroll=8)
    one16 = jnp.ones((LANES,), jnp.float32)

    def body(i, carry):
        idx = idx_v[pl.ds(i * LANES, LANES)]
        plsc.addupdate_scatter(deg_v, [idx], one16)
        return carry

    lax.fori_loop(0, EPT_RAW // LANES, body, 0, unroll=5)
    pltpu.sync_copy(deg_v, out_hbm.at[wid])


NBUF = 4
CPT = EPT // CHUNK


@functools.partial(
    pl.kernel,
    out_type=jax.ShapeDtypeStruct((NC, NPAD, D_HID), jnp.float32),
    mesh=_mesh,
    scratch_types=[
        pltpu.VMEM((EPT,), jnp.int32),
        pltpu.VMEM((CPT, CHUNK), jnp.int32),
        pltpu.VMEM((NBUF, CHUNK, D_HID), jnp.float32),
        pltpu.VMEM_SHARED((NPAD, D_HID), jnp.float32),
    ]
    + [pltpu.SemaphoreType.DMA] * NBUF,
    compiler_params=pltpu.CompilerParams(use_tc_tiling_on_sc=False),
)
def _sc_scatter(src_hbm, dst_hbm, hn_hbm, z_hbm, out_hbm,
                srcv, dstv, rows, agg, *sems):
    cid = lax.axis_index("c")
    sid = lax.axis_index("s")
    wid = sid * NC + cid
    pltpu.sync_copy(src_hbm.at[pl.ds(wid * EPT, EPT)], srcv)
    pltpu.sync_copy(dst_hbm.at[pl.ds(wid * CPT, CPT)], dstv)

    def sidx(j):
        return srcv.at[pl.ds(j * CHUNK, CHUNK)]
    stripe = pl.ds(sid * ROWS_PT, ROWS_PT)

    @pl.when(cid == 0)
    def _():
        pltpu.sync_copy(hn_hbm.at[stripe], agg.at[stripe])

    @pl.when(cid == 1)
    def _():
        pltpu.sync_copy(z_hbm, agg.at[stripe])

    for b in range(NBUF - 1):
        pltpu.async_copy(hn_hbm.at[sidx(b)], rows.at[b], sems[b])
    plsc.subcore_barrier()

    def group(g, carry):
        j0 = g * NBUF
        for b in range(NBUF):
            j = j0 + b
            jn = j + NBUF - 1
            bn = (b - 1) % NBUF

            @pl.when(jn < CPT)
            def _():
                pltpu.async_copy(hn_hbm.at[sidx(jn)], rows.at[bn], sems[bn])

            pltpu.make_async_copy(hn_hbm.at[sidx(0)], rows.at[b], sems[b]).wait()
            pltpu.sync_copy(rows.at[b], agg.at[dstv.at[j]], add=True)
        return carry

    lax.fori_loop(0, CPT // NBUF, group, 0)
    plsc.subcore_barrier()
    pltpu.sync_copy(agg.at[pl.ds(sid * ROWS_PT, ROWS_PT)],
                    out_hbm.at[cid, pl.ds(sid * ROWS_PT, ROWS_PT)])


def _tc_hn_body(degp_ref, x_ref, w_ref, hn_ref, dis_ref):
    ones = jnp.ones((NW, 1), jnp.float32)
    deg_col = lax.dot_general(degp_ref[...], ones, (((0,), (0,)), ((), ())),
                              preferred_element_type=jnp.float32) + 1.0
    dis = lax.rsqrt(deg_col)
    h = jnp.dot(x_ref[...], w_ref[...], preferred_element_type=jnp.float32)
    hn_ref[:N_NODES] = h * dis[:N_NODES]
    hn_ref[N_NODES:] = jnp.zeros((NPAD - N_NODES, D_HID), jnp.float32)
    dis_ref[...] = dis


BLK_D = 2000


def _tc_final_body(p_ref, dis_ref, w2_ref, b1_ref, b2_ref, o_ref):
    s = p_ref[0] + p_ref[1]
    a = s * dis_ref[...] + b1_ref[...][None, :]
    r = jnp.maximum(a, 0.0)
    o = jnp.dot(r, w2_ref[...], preferred_element_type=jnp.float32)
    o = o + b2_ref[...][None, :]
    m = jnp.max(o, axis=-1, keepdims=True)
    ex = jnp.exp(o - m)
    lse = jnp.log(jnp.sum(ex, axis=-1, keepdims=True)) + m
    o_ref[...] = o - lse


def kernel(x, edge_index, batch, W1, b1, W2, b2):
    del batch
    src = edge_index[0].astype(jnp.int32)
    dst = edge_index[1].astype(jnp.int32)
    n_pad_e = E_PAD - N_EDGES
    pad_idx = N_NODES + (jnp.arange(n_pad_e, dtype=jnp.int32) % (NPAD - N_NODES))
    src_p2 = jnp.concatenate([src, pad_idx])
    dst_p2 = jnp.concatenate([dst, pad_idx]).reshape(NW * (EPT // CHUNK), CHUNK)

    degp = _sc_degree(dst)

    hn, dis_col = pl.pallas_call(
        _tc_hn_body,
        out_shape=(jax.ShapeDtypeStruct((NPAD, D_HID), jnp.float32),
                   jax.ShapeDtypeStruct((NPAD, 1), jnp.float32)),
    )(degp, x, W1)

    zeros = jnp.zeros((ROWS_PT, D_HID), jnp.float32)
    p = _sc_scatter(src_p2, dst_p2, hn, zeros)

    out = pl.pallas_call(
        _tc_final_body,
        grid=(N_NODES // BLK_D,),
        in_specs=[
            pl.BlockSpec((NC, BLK_D, D_HID), lambda i: (0, i, 0)),
            pl.BlockSpec((BLK_D, 1), lambda i: (i, 0)),
            pl.BlockSpec((D_HID, D_OUT), lambda i: (0, 0)),
            pl.BlockSpec((D_HID,), lambda i: (0,)),
            pl.BlockSpec((D_OUT,), lambda i: (0,)),
        ],
        out_specs=pl.BlockSpec((BLK_D, D_OUT), lambda i: (i, 0)),
        out_shape=jax.ShapeDtypeStruct((N_NODES, D_OUT), jnp.float32),
    )(p, dis_col, W2, b1, b2)
    return out


# speedup vs baseline: 1.0775x; 1.0694x over previous
import functools

import jax
import jax.numpy as jnp
from jax import lax
from jax.experimental import pallas as pl
from jax.experimental.pallas import tpu as pltpu
from jax.experimental.pallas import tpu_sc as plsc

N_NODES = 10000
N_EDGES = 320000
D_IN = 128
D_HID = 128
D_OUT = 40

NC = 2
NS = 16
NW = NC * NS
LANES = 16

NPAD = 10112
CHUNK = 32
EPT = 10240
E_PAD = NW * EPT
EPT_RAW = N_EDGES // NW
ROWS_PT = NPAD // NS

_mesh = plsc.VectorSubcoreMesh(
    core_axis_name="c", subcore_axis_name="s", num_cores=NC, num_subcores=NS
)


@functools.partial(
    pl.kernel,
    out_type=jax.ShapeDtypeStruct((NW, NPAD), jnp.float32),
    mesh=_mesh,
    scratch_types=[
        pltpu.VMEM((EPT_RAW,), jnp.int32),
        pltpu.VMEM((NPAD,), jnp.float32),
    ],
    compiler_params=pltpu.CompilerParams(needs_layout_passes=False),
)
def _sc_degree(dst_hbm, out_hbm, idx_v, deg_v):
    wid = lax.axis_index("s") * NC + lax.axis_index("c")
    pltpu.sync_copy(dst_hbm.at[pl.ds(wid * EPT_RAW, EPT_RAW)], idx_v)
    zero16 = jnp.zeros((LANES,), jnp.float32)

    def zbody(i, carry):
        deg_v[pl.ds(i * LANES, LANES)] = zero16
        return carry

    lax.fori_loop(0, NPAD // LANES, zbody, 0, unroll=8)
    one16 = jnp.ones((LANES,), jnp.float32)

    def body(i, carry):
        idx = idx_v[pl.ds(i * LANES, LANES)]
        plsc.addupdate_scatter(deg_v, [idx], one16)
        return carry

    lax.fori_loop(0, EPT_RAW // LANES, body, 0, unroll=5)
    pltpu.sync_copy(deg_v, out_hbm.at[wid])


NBUF = 5
CPT = EPT // CHUNK


@functools.partial(
    pl.kernel,
    out_type=jax.ShapeDtypeStruct((NC, NPAD, D_HID), jnp.float32),
    mesh=_mesh,
    scratch_types=[
        pltpu.VMEM((EPT,), jnp.int32),
        pltpu.VMEM((CPT, CHUNK), jnp.int32),
        pltpu.VMEM((NBUF, CHUNK, D_HID), jnp.float32),
        pltpu.VMEM_SHARED((NPAD, D_HID), jnp.float32),
    ]
    + [pltpu.SemaphoreType.DMA] * NBUF,
    compiler_params=pltpu.CompilerParams(use_tc_tiling_on_sc=False),
)
def _sc_scatter(src_hbm, dst_hbm, hn_hbm, z_hbm, out_hbm,
                srcv, dstv, rows, agg, *sems):
    cid = lax.axis_index("c")
    sid = lax.axis_index("s")
    wid = sid * NC + cid
    pltpu.sync_copy(src_hbm.at[pl.ds(wid * EPT, EPT)], srcv)
    pltpu.sync_copy(dst_hbm.at[pl.ds(wid * CPT, CPT)], dstv)

    def sidx(j):
        return srcv.at[pl.ds(j * CHUNK, CHUNK)]
    stripe = pl.ds(sid * ROWS_PT, ROWS_PT)

    @pl.when(cid == 0)
    def _():
        pltpu.sync_copy(hn_hbm.at[stripe], agg.at[stripe])

    @pl.when(cid == 1)
    def _():
        pltpu.sync_copy(z_hbm, agg.at[stripe])

    for b in range(NBUF - 1):
        pltpu.async_copy(hn_hbm.at[sidx(b)], rows.at[b], sems[b])
    plsc.subcore_barrier()

    def group(g, carry):
        j0 = g * NBUF
        for b in range(NBUF):
            j = j0 + b
            jn = j + NBUF - 1
            bn = (b - 1) % NBUF

            @pl.when(jn < CPT)
            def _():
                pltpu.async_copy(hn_hbm.at[sidx(jn)], rows.at[bn], sems[bn])

            pltpu.make_async_copy(hn_hbm.at[sidx(0)], rows.at[b], sems[b]).wait()
            pltpu.sync_copy(rows.at[b], agg.at[dstv.at[j]], add=True)
        return carry

    lax.fori_loop(0, CPT // NBUF, group, 0)
    plsc.subcore_barrier()
    pltpu.sync_copy(agg.at[pl.ds(sid * ROWS_PT, ROWS_PT)],
                    out_hbm.at[cid, pl.ds(sid * ROWS_PT, ROWS_PT)])


def _tc_hn_body(degp_ref, x_ref, w_ref, hn_ref, dis_ref):
    ones = jnp.ones((NW, 1), jnp.float32)
    deg_col = lax.dot_general(degp_ref[...], ones, (((0,), (0,)), ((), ())),
                              preferred_element_type=jnp.float32) + 1.0
    dis = lax.rsqrt(deg_col)
    h = jnp.dot(x_ref[...], w_ref[...], preferred_element_type=jnp.float32)
    hn_ref[:N_NODES] = h * dis[:N_NODES]
    hn_ref[N_NODES:] = jnp.zeros((NPAD - N_NODES, D_HID), jnp.float32)
    dis_ref[...] = dis


BLK_D = 2000


def _tc_final_body(p_ref, dis_ref, w2_ref, b1_ref, b2_ref, o_ref):
    s = p_ref[0] + p_ref[1]
    a = s * dis_ref[...] + b1_ref[...][None, :]
    r = jnp.maximum(a, 0.0)
    o = jnp.dot(r, w2_ref[...], preferred_element_type=jnp.float32)
    o = o + b2_ref[...][None, :]
    m = jnp.max(o, axis=-1, keepdims=True)
    ex = jnp.exp(o - m)
    lse = jnp.log(jnp.sum(ex, axis=-1, keepdims=True)) + m
    o_ref[...] = o - lse


def kernel(x, edge_index, batch, W1, b1, W2, b2):
    del batch
    src = edge_index[0].astype(jnp.int32)
    dst = edge_index[1].astype(jnp.int32)
    n_pad_e = E_PAD - N_EDGES
    pad_idx = N_NODES + (jnp.arange(n_pad_e, dtype=jnp.int32) % (NPAD - N_NODES))
    src_p2 = jnp.concatenate([src, pad_idx])
    dst_p2 = jnp.concatenate([dst, pad_idx]).reshape(NW * (EPT // CHUNK), CHUNK)

    degp = _sc_degree(dst)

    hn, dis_col = pl.pallas_call(
        _tc_hn_body,
        out_shape=(jax.ShapeDtypeStruct((NPAD, D_HID), jnp.float32),
                   jax.ShapeDtypeStruct((NPAD, 1), jnp.float32)),
    )(degp, x, W1)

    zeros = jnp.zeros((ROWS_PT, D_HID), jnp.float32)
    p = _sc_scatter(src_p2, dst_p2, hn, zeros)

    out = pl.pallas_call(
        _tc_final_body,
        grid=(N_NODES // BLK_D,),
        in_specs=[
            pl.BlockSpec((NC, BLK_D, D_HID), lambda i: (0, i, 0)),
            pl.BlockSpec((BLK_D, 1), lambda i: (i, 0)),
            pl.BlockSpec((D_HID, D_OUT), lambda i: (0, 0)),
            pl.BlockSpec((D_HID,), lambda i: (0,)),
            pl.BlockSpec((D_OUT,), lambda i: (0,)),
        ],
        out_specs=pl.BlockSpec((BLK_D, D_OUT), lambda i: (i, 0)),
        out_shape=jax.ShapeDtypeStruct((N_NODES, D_OUT), jnp.float32),
    )(p, dis_col, W2, b1, b2)
    return out
